# baseline (speedup 1.0000x reference)
import functools

import jax
import jax.numpy as jnp
from jax import lax
from jax.experimental import pallas as pl
from jax.experimental.pallas import tpu as pltpu
from jax.experimental.pallas import tpu_sc as plsc

BG, MD, KL, EL, DL, RL = 5, 4, 3, 3, 3, 4
DK, DE, DD, DR = BG * MD * KL, BG * MD * EL, BG * DL, RL
NROW = 101

_mesh = plsc.VectorSubcoreMesh(core_axis_name="c", subcore_axis_name="s")


def _pad16(n):
    return ((n + 15) // 16) * 16


@functools.partial(
    pl.kernel,
    out_type=(
        jax.ShapeDtypeStruct((DK,), jnp.float32),
        jax.ShapeDtypeStruct((DE,), jnp.float32),
        jax.ShapeDtypeStruct((DD,), jnp.float32),
        jax.ShapeDtypeStruct((DR,), jnp.float32),
    ),
    mesh=_mesh,
    compiler_params=pltpu.CompilerParams(needs_layout_passes=False),
    scratch_types=[
        pltpu.VMEM((16,), jnp.float32),
        pltpu.VMEM((NROW, DK), jnp.float32),
        pltpu.VMEM((NROW, DE), jnp.float32),
        pltpu.VMEM((NROW, DD), jnp.float32),
        pltpu.VMEM((NROW, DR), jnp.float32),
        pltpu.VMEM((_pad16(DK),), jnp.float32),
        pltpu.VMEM((_pad16(DE),), jnp.float32),
        pltpu.VMEM((_pad16(DD),), jnp.float32),
        pltpu.VMEM((_pad16(DR),), jnp.float32),
        pltpu.SemaphoreType.DMA,
    ],
)
def _lookup(x_hbm, wk_hbm, we_hbm, wd_hbm, wr_hbm,
            ok_hbm, oe_hbm, od_hbm, or_hbm,
            x_v, wk_v, we_v, wd_v, wr_v,
            bk_v, be_v, bd_v, br_v, sem):
    is_leader = jnp.logical_and(lax.axis_index("c") == 0, lax.axis_index("s") == 0)

    @pl.when(is_leader)
    def _():
        cps = [
            pltpu.async_copy(x_hbm, x_v.at[pl.ds(0, 1)], sem),
            pltpu.async_copy(wk_hbm, wk_v, sem),
            pltpu.async_copy(we_hbm, we_v, sem),
            pltpu.async_copy(wd_hbm, wd_v, sem),
            pltpu.async_copy(wr_hbm, wr_v, sem),
        ]
        for cp in cps:
            cp.wait()
        v100 = x_v[...][0] * 100.0
        idx0 = v100.astype(jnp.int32)
        idx = idx0 - (idx0.astype(jnp.float32) > v100).astype(jnp.int32)
        row = jnp.full((16,), idx, dtype=jnp.int32)
        lanes = lax.iota(jnp.int32, 16)
        for tab_v, buf_v, width in (
            (wk_v, bk_v, DK),
            (we_v, be_v, DE),
            (wd_v, bd_v, DD),
            (wr_v, br_v, DR),
        ):
            for j in range(_pad16(width) // 16):
                cols = jnp.minimum(lanes + (j * 16), width - 1)
                buf_v[pl.ds(j * 16, 16)] = plsc.load_gather(tab_v, [row, cols])
        ocps = [
            pltpu.async_copy(bk_v.at[pl.ds(0, DK)], ok_hbm, sem),
            pltpu.async_copy(be_v.at[pl.ds(0, DE)], oe_hbm, sem),
            pltpu.async_copy(bd_v.at[pl.ds(0, DD)], od_hbm, sem),
            pltpu.async_copy(br_v.at[pl.ds(0, DR)], or_hbm, sem),
        ]
        for cp in ocps:
            cp.wait()


def kernel(x, W_kernel, W_expand, W_depth, W_res):
    ok, oe, od, orr = _lookup(x.reshape(1), W_kernel, W_expand, W_depth, W_res)
    return (
        ok.reshape(BG, MD, KL),
        oe.reshape(BG, MD, EL),
        od.reshape(BG, DL),
        orr.reshape(1, RL),
    )


# speedup vs baseline: 1.0736x; 1.0736x over previous
import functools

import jax
import jax.numpy as jnp
from jax import lax
from jax.experimental import pallas as pl
from jax.experimental.pallas import tpu as pltpu
from jax.experimental.pallas import tpu_sc as plsc

BG, MD, KL, EL, DL, RL = 5, 4, 3, 3, 3, 4
DK, DE, DD, DR = BG * MD * KL, BG * MD * EL, BG * DL, RL
NROW = 101

_mesh = plsc.VectorSubcoreMesh(core_axis_name="c", subcore_axis_name="s", num_cores=1)


def _pad16(n):
    return ((n + 15) // 16) * 16


@functools.partial(
    pl.kernel,
    out_type=(
        jax.ShapeDtypeStruct((DK,), jnp.float32),
        jax.ShapeDtypeStruct((DE,), jnp.float32),
        jax.ShapeDtypeStruct((DD,), jnp.float32),
        jax.ShapeDtypeStruct((DR,), jnp.float32),
    ),
    mesh=_mesh,
    compiler_params=pltpu.CompilerParams(needs_layout_passes=False),
    scratch_types=[
        pltpu.VMEM((16,), jnp.float32),
        pltpu.VMEM((NROW, DK), jnp.float32),
        pltpu.VMEM((NROW, DE), jnp.float32),
        pltpu.VMEM((NROW, DD), jnp.float32),
        pltpu.VMEM((NROW, DR), jnp.float32),
        pltpu.VMEM((_pad16(DK),), jnp.float32),
        pltpu.VMEM((_pad16(DE),), jnp.float32),
        pltpu.VMEM((_pad16(DD),), jnp.float32),
        pltpu.VMEM((_pad16(DR),), jnp.float32),
        pltpu.SemaphoreType.DMA,
    ],
)
def _lookup(x_hbm, wk_hbm, we_hbm, wd_hbm, wr_hbm,
            ok_hbm, oe_hbm, od_hbm, or_hbm,
            x_v, wk_v, we_v, wd_v, wr_v,
            bk_v, be_v, bd_v, br_v, sem):
    is_leader = jnp.logical_and(lax.axis_index("c") == 0, lax.axis_index("s") == 0)

    @pl.when(is_leader)
    def _():
        cps = [
            pltpu.async_copy(x_hbm, x_v.at[pl.ds(0, 1)], sem),
            pltpu.async_copy(wk_hbm, wk_v, sem),
            pltpu.async_copy(we_hbm, we_v, sem),
            pltpu.async_copy(wd_hbm, wd_v, sem),
            pltpu.async_copy(wr_hbm, wr_v, sem),
        ]
        for cp in cps:
            cp.wait()
        v100 = x_v[...][0] * 100.0
        idx0 = v100.astype(jnp.int32)
        idx = idx0 - (idx0.astype(jnp.float32) > v100).astype(jnp.int32)
        row = jnp.full((16,), idx, dtype=jnp.int32)
        lanes = lax.iota(jnp.int32, 16)
        for tab_v, buf_v, width in (
            (wk_v, bk_v, DK),
            (we_v, be_v, DE),
            (wd_v, bd_v, DD),
            (wr_v, br_v, DR),
        ):
            for j in range(_pad16(width) // 16):
                cols = jnp.minimum(lanes + (j * 16), width - 1)
                buf_v[pl.ds(j * 16, 16)] = plsc.load_gather(tab_v, [row, cols])
        ocps = [
            pltpu.async_copy(bk_v.at[pl.ds(0, DK)], ok_hbm, sem),
            pltpu.async_copy(be_v.at[pl.ds(0, DE)], oe_hbm, sem),
            pltpu.async_copy(bd_v.at[pl.ds(0, DD)], od_hbm, sem),
            pltpu.async_copy(br_v.at[pl.ds(0, DR)], or_hbm, sem),
        ]
        for cp in ocps:
            cp.wait()


def kernel(x, W_kernel, W_expand, W_depth, W_res):
    ok, oe, od, orr = _lookup(x.reshape(1), W_kernel, W_expand, W_depth, W_res)
    return (
        ok.reshape(BG, MD, KL),
        oe.reshape(BG, MD, EL),
        od.reshape(BG, DL),
        orr.reshape(1, RL),
    )
